# baseline (speedup 1.0000x reference)
import functools

import jax
import jax.numpy as jnp
from jax import lax
from jax.experimental import pallas as pl
from jax.experimental.pallas import tpu as pltpu
from jax.experimental.pallas import tpu_sc as plsc

N_ATOMS = 10000
K_NBR = 16
H_A = 64
H_P = 2 * H_A
H_B = 101
H_BP = 128
N_LAYERS = 3
BATCH = 2
N_CLASSES = 5
STEP = 0.1
INV_STEP2 = 1.0 / (STEP * STEP)

ROWS = BATCH * N_ATOMS
EDGES = ROWS * K_NBR

BN = 400
NBN = N_ATOMS // BN

N_WORKERS = 32
PER_W = EDGES // N_WORKERS
CHUNK = 80
N_STEPS = PER_W // CHUNK


def _h0_body(emb_ref, w_ref, h_ref):
    h = jnp.dot(emb_ref[...], w_ref[...], preferred_element_type=jnp.float32)
    h_ref[...] = jnp.concatenate([h, h], axis=1)


def _make_h0(interpret=False):
    return pl.pallas_call(
        _h0_body,
        grid=(NBN,),
        in_specs=[
            pl.BlockSpec((BN, H_A), lambda i: (i, 0)),
            pl.BlockSpec((H_A, H_A), lambda i: (0, 0)),
        ],
        out_specs=pl.BlockSpec((BN, H_P), lambda i: (i, 0)),
        out_shape=jax.ShapeDtypeStruct((N_ATOMS, H_P), jnp.float32),
        interpret=interpret,
    )


N_PAIRS = N_STEPS // 2
TAIL_C = N_STEPS - 1


def _sc_gather_body(table_ref, idx_ref, out_ref, idx_a, idx_b, rows_a, rows_b,
                    si_a, si_b, sg_a, sg_b, so_a, so_b):
    wid = lax.axis_index("s") * 2 + lax.axis_index("c")
    base = wid * PER_W

    def idx_sl(c):
        return idx_ref.at[pl.ds(pl.multiple_of(base + c * CHUNK, 8), CHUNK)]

    def out_sl(c):
        return out_ref.at[pl.ds(pl.multiple_of(base + c * CHUNK, 8), CHUNK)]

    pltpu.async_copy(idx_sl(0), idx_a, si_a)
    pltpu.async_copy(idx_sl(1), idx_b, si_b)

    def pair(j, carry):
        c0 = 2 * j
        c1 = c0 + 1

        @pl.when(j > 0)
        def _wait_prev_out():
            pltpu.make_async_copy(rows_a, out_sl(c0), so_a).wait()
            pltpu.make_async_copy(rows_b, out_sl(c1), so_b).wait()

        pltpu.make_async_copy(idx_sl(c0), idx_a, si_a).wait()
        ga = pltpu.async_copy(table_ref.at[idx_a], rows_a, sg_a)
        pltpu.make_async_copy(idx_sl(c1), idx_b, si_b).wait()
        gb = pltpu.async_copy(table_ref.at[idx_b], rows_b, sg_b)
        ga.wait()
        pltpu.async_copy(rows_a, out_sl(c0), so_a)
        gb.wait()
        pltpu.async_copy(rows_b, out_sl(c1), so_b)

        @pl.when(j < N_PAIRS - 1)
        def _prefetch():
            pltpu.async_copy(idx_sl(c0 + 2), idx_a, si_a)
            pltpu.async_copy(idx_sl(c1 + 2), idx_b, si_b)

        @pl.when(j == N_PAIRS - 1)
        def _prefetch_tail():
            pltpu.async_copy(idx_sl(TAIL_C), idx_a, si_a)

        return carry

    lax.fori_loop(0, N_PAIRS, pair, 0)

    pltpu.make_async_copy(rows_a, out_sl(TAIL_C), so_a).wait()
    pltpu.make_async_copy(idx_sl(TAIL_C), idx_a, si_a).wait()
    pltpu.async_copy(table_ref.at[idx_a], rows_a, sg_a).wait()
    pltpu.async_copy(rows_a, out_sl(TAIL_C), so_a)
    pltpu.make_async_copy(rows_a, out_sl(TAIL_C), so_a).wait()
    pltpu.make_async_copy(rows_b, out_sl(TAIL_C - 1), so_b).wait()


def _make_sc_gather():
    mesh = plsc.VectorSubcoreMesh(core_axis_name="c", subcore_axis_name="s",
                                  num_cores=2, num_subcores=16)
    return pl.kernel(
        _sc_gather_body,
        out_type=jax.ShapeDtypeStruct((EDGES, H_P), jnp.float32),
        mesh=mesh,
        scratch_types=[
            pltpu.VMEM((CHUNK,), jnp.int32),
            pltpu.VMEM((CHUNK,), jnp.int32),
            pltpu.VMEM((CHUNK, H_P), jnp.float32),
            pltpu.VMEM((CHUNK, H_P), jnp.float32),
            pltpu.SemaphoreType.DMA,
            pltpu.SemaphoreType.DMA,
            pltpu.SemaphoreType.DMA,
            pltpu.SemaphoreType.DMA,
            pltpu.SemaphoreType.DMA,
            pltpu.SemaphoreType.DMA,
        ],
    )


def _combine_body(layer0, has_next, *refs):
    if has_next:
        (d0_ref, d1_ref, nf_ref, x_ref, wf1, bf1, wf2, bf2, wat,
         wo1, bo1, wo2, bo2, wnext, xo_ref, ho_ref) = refs
    else:
        (d0_ref, d1_ref, nf_ref, x_ref, wf1, bf1, wf2, bf2, wat,
         wo1, bo1, wo2, bo2, xo_ref) = refs

    filt = STEP * lax.broadcasted_iota(jnp.int32, (1, H_BP), 1).astype(
        jnp.float32)

    d_parts = []
    nf_parts = []
    for b, dist_ref in ((0, d0_ref), (1, d1_ref)):
        for k in range(K_NBR):
            d_parts.append(dist_ref[:, k:k + 1])
            nf_parts.append(nf_ref[2 * k + b, :, H_A * b:H_A * (b + 1)])
    d_all = jnp.concatenate(d_parts, axis=0)
    nf_all = jnp.concatenate(nf_parts, axis=0)

    g = jnp.exp(-((d_all - filt) ** 2) * INV_STEP2)
    t1 = jnp.tanh(jnp.dot(g.astype(jnp.bfloat16), wf1[...],
                          preferred_element_type=jnp.float32) + bf1[...])
    f = jnp.dot(t1.astype(jnp.bfloat16), wf2[...],
                preferred_element_type=jnp.float32) + bf2[...]
    cf = nf_all * f
    s = jnp.sum(cf * wat[...], axis=1, keepdims=True)

    aggs = []
    for b in range(2):
        sb = jnp.concatenate(
            [s[(b * K_NBR + k) * BN:(b * K_NBR + k + 1) * BN]
             for k in range(K_NBR)], axis=1)
        mx = jnp.max(sb, axis=1, keepdims=True)
        e = jnp.exp(sb - mx)
        attn = e / jnp.sum(e, axis=1, keepdims=True)
        agg = jnp.zeros((BN, H_A), dtype=jnp.float32)
        for k in range(K_NBR):
            j = b * K_NBR + k
            agg += attn[:, k:k + 1] * cf[j * BN:(j + 1) * BN]
        aggs.append(agg)
    agg2 = jnp.concatenate(aggs, axis=0)

    t2 = jnp.tanh(jnp.dot(agg2, wo1[...],
                          preferred_element_type=jnp.float32) + bo1[...])
    out = jnp.dot(t2, wo2[...],
                  preferred_element_type=jnp.float32) + bo2[...]
    if layer0:
        xv = x_ref[...]
        x2 = jnp.concatenate([xv, xv], axis=0)
    else:
        x2 = jnp.concatenate([x_ref[:, :H_A], x_ref[:, H_A:]], axis=0)
    xn = x2 + out
    xo_ref[...] = jnp.concatenate([xn[:BN], xn[BN:]], axis=1)
    if has_next:
        hn = jnp.dot(xn, wnext[...], preferred_element_type=jnp.float32)
        ho_ref[...] = jnp.concatenate([hn[:BN], hn[BN:]], axis=1)


def _make_combine(layer0, has_next, interpret=False):
    if layer0:
        x_spec = pl.BlockSpec((BN, H_A), lambda i: (i, 0))
    else:
        x_spec = pl.BlockSpec((BN, H_P), lambda i: (i, 0))
    full = lambda r, c: pl.BlockSpec((r, c), lambda i: (0, 0))
    in_specs = [
        pl.BlockSpec((BN, K_NBR), lambda i: (i, 0)),
        pl.BlockSpec((BN, K_NBR), lambda i: (i + NBN, 0)),
        pl.BlockSpec((2 * K_NBR, BN, H_P), lambda i: (0, i, 0)),
        x_spec,
        full(H_BP, H_A),
        full(1, H_A),
        full(H_A, H_A),
        full(1, H_A),
        full(1, H_A),
        full(H_A, H_A),
        full(1, H_A),
        full(H_A, H_A),
        full(1, H_A),
    ]
    out_shapes = [jax.ShapeDtypeStruct((N_ATOMS, H_P), jnp.float32)]
    out_specs = [pl.BlockSpec((BN, H_P), lambda i: (i, 0))]
    if has_next:
        in_specs.append(full(H_A, H_A))
        out_shapes.append(jax.ShapeDtypeStruct((N_ATOMS, H_P), jnp.float32))
        out_specs.append(pl.BlockSpec((BN, H_P), lambda i: (i, 0)))
    return pl.pallas_call(
        functools.partial(_combine_body, layer0, has_next),
        grid=(NBN,),
        in_specs=in_specs,
        out_specs=out_specs,
        out_shape=out_shapes,
        interpret=interpret,
    )


def _head_body(x_ref, wa, ba, wc, bc, out_ref, acc_ref):
    i = pl.program_id(0)

    @pl.when(i == 0)
    def _init():
        acc_ref[...] = jnp.zeros_like(acc_ref)

    acc_ref[...] += jnp.sum(x_ref[...], axis=0, keepdims=True)

    @pl.when(i == NBN - 1)
    def _fin():
        accv = acc_ref[...] * (1.0 / N_ATOMS)
        pooled = jnp.concatenate([accv[:, :H_A], accv[:, H_A:]], axis=0)
        emb = jnp.dot(pooled, wa[...],
                      preferred_element_type=jnp.float32) + ba[...]
        logits = jnp.dot(emb, wc[...],
                         preferred_element_type=jnp.float32) + bc[...]
        mx = jnp.max(logits, axis=1, keepdims=True)
        e = jnp.exp(logits - mx)
        out_ref[...] = e / jnp.sum(e, axis=1, keepdims=True)


def _make_head(interpret=False):
    full = lambda r, c: pl.BlockSpec((r, c), lambda i: (0, 0))
    return pl.pallas_call(
        _head_body,
        grid=(NBN,),
        in_specs=[
            pl.BlockSpec((BN, H_P), lambda i: (i, 0)),
            full(H_A, 32),
            full(1, 32),
            full(32, N_CLASSES),
            full(1, N_CLASSES),
        ],
        out_specs=full(BATCH, N_CLASSES),
        out_shape=jax.ShapeDtypeStruct((BATCH, N_CLASSES), jnp.float32),
        scratch_shapes=[pltpu.VMEM((1, H_P), jnp.float32)],
        interpret=interpret,
    )


def kernel(data, atom_embeddings, conv_params, W_amino, b_amino, W_cls, b_cls):
    dist = data[:, :, :K_NBR].reshape(ROWS, K_NBR)
    idx = data[:, :, K_NBR:].astype(jnp.int32).transpose(2, 0, 1).reshape(
        EDGES)

    h = _make_h0()(atom_embeddings, conv_params[0]['W_init'])
    sc_gather = _make_sc_gather()

    x = atom_embeddings
    for l in range(N_LAYERS):
        p = conv_params[l]
        nf = sc_gather(h, idx).reshape(2 * K_NBR, N_ATOMS, H_P)
        wf1p = jnp.concatenate(
            [p['W_f1'], jnp.zeros((H_BP - H_B, H_A), jnp.float32)],
            axis=0).astype(jnp.bfloat16)
        args = [dist, dist, nf, x, wf1p,
                p['b_f1'].reshape(1, H_A), p['W_f2'].astype(jnp.bfloat16),
                p['b_f2'].reshape(1, H_A), p['w_attn'].reshape(1, H_A),
                p['W_o1'], p['b_o1'].reshape(1, H_A),
                p['W_o2'], p['b_o2'].reshape(1, H_A)]
        has_next = l < N_LAYERS - 1
        if has_next:
            args.append(conv_params[l + 1]['W_init'])
            x, h = _make_combine(l == 0, True)(*args)
        else:
            (x,) = _make_combine(l == 0, False)(*args)

    return _make_head()(x, W_amino, b_amino.reshape(1, 32),
                        W_cls, b_cls.reshape(1, N_CLASSES))


# speedup vs baseline: 1.2029x; 1.2029x over previous
import functools

import jax
import jax.numpy as jnp
from jax import lax
from jax.experimental import pallas as pl
from jax.experimental.pallas import tpu as pltpu
from jax.experimental.pallas import tpu_sc as plsc

N_ATOMS = 10000
K_NBR = 16
H_A = 64
H_P = 2 * H_A
H_B = 101
H_BP = 128
N_LAYERS = 3
BATCH = 2
N_CLASSES = 5
STEP = 0.1
INV_STEP2 = 1.0 / (STEP * STEP)

ROWS = BATCH * N_ATOMS
EDGES = ROWS * K_NBR

BN = 400
NBN = N_ATOMS // BN

N_WORKERS = 32
PER_W = EDGES // N_WORKERS
CHUNK = 80
N_STEPS = PER_W // CHUNK


def _h0_body(emb_ref, w_ref, h_ref):
    h = jnp.dot(emb_ref[...], w_ref[...], preferred_element_type=jnp.float32)
    h_ref[...] = jnp.concatenate([h, h], axis=1)


def _make_h0(interpret=False):
    return pl.pallas_call(
        _h0_body,
        grid=(NBN,),
        in_specs=[
            pl.BlockSpec((BN, H_A), lambda i: (i, 0)),
            pl.BlockSpec((H_A, H_A), lambda i: (0, 0)),
        ],
        out_specs=pl.BlockSpec((BN, H_P), lambda i: (i, 0)),
        out_shape=jax.ShapeDtypeStruct((N_ATOMS, H_P), jnp.float32),
        interpret=interpret,
    )


N_PAIRS = N_STEPS // 2
TAIL_C = N_STEPS - 1


def _sc_gather_body(table_ref, idx_ref, out_ref, idx_a, idx_b, rows_a, rows_b,
                    si_a, si_b, sg_a, sg_b, so_a, so_b):
    wid = lax.axis_index("s") * 2 + lax.axis_index("c")
    base = wid * PER_W

    def idx_sl(c):
        return idx_ref.at[pl.ds(pl.multiple_of(base + c * CHUNK, 8), CHUNK)]

    def out_sl(c):
        return out_ref.at[pl.ds(pl.multiple_of(base + c * CHUNK, 8), CHUNK)]

    pltpu.async_copy(idx_sl(0), idx_a, si_a)
    pltpu.async_copy(idx_sl(1), idx_b, si_b)

    def pair(j, carry):
        c0 = 2 * j
        c1 = c0 + 1

        @pl.when(j > 0)
        def _wait_prev_out():
            pltpu.make_async_copy(rows_a, out_sl(c0), so_a).wait()
            pltpu.make_async_copy(rows_b, out_sl(c1), so_b).wait()

        pltpu.make_async_copy(idx_sl(c0), idx_a, si_a).wait()
        ga = pltpu.async_copy(table_ref.at[idx_a], rows_a, sg_a)
        pltpu.make_async_copy(idx_sl(c1), idx_b, si_b).wait()
        gb = pltpu.async_copy(table_ref.at[idx_b], rows_b, sg_b)
        ga.wait()
        pltpu.async_copy(rows_a, out_sl(c0), so_a)
        gb.wait()
        pltpu.async_copy(rows_b, out_sl(c1), so_b)

        @pl.when(j < N_PAIRS - 1)
        def _prefetch():
            pltpu.async_copy(idx_sl(c0 + 2), idx_a, si_a)
            pltpu.async_copy(idx_sl(c1 + 2), idx_b, si_b)

        @pl.when(j == N_PAIRS - 1)
        def _prefetch_tail():
            pltpu.async_copy(idx_sl(TAIL_C), idx_a, si_a)

        return carry

    lax.fori_loop(0, N_PAIRS, pair, 0)

    pltpu.make_async_copy(rows_a, out_sl(TAIL_C), so_a).wait()
    pltpu.make_async_copy(idx_sl(TAIL_C), idx_a, si_a).wait()
    pltpu.async_copy(table_ref.at[idx_a], rows_a, sg_a).wait()
    pltpu.async_copy(rows_a, out_sl(TAIL_C), so_a)
    pltpu.make_async_copy(rows_a, out_sl(TAIL_C), so_a).wait()
    pltpu.make_async_copy(rows_b, out_sl(TAIL_C - 1), so_b).wait()


def _make_sc_gather():
    mesh = plsc.VectorSubcoreMesh(core_axis_name="c", subcore_axis_name="s",
                                  num_cores=2, num_subcores=16)
    return pl.kernel(
        _sc_gather_body,
        out_type=jax.ShapeDtypeStruct((EDGES, H_P), jnp.float32),
        mesh=mesh,
        scratch_types=[
            pltpu.VMEM((CHUNK,), jnp.int32),
            pltpu.VMEM((CHUNK,), jnp.int32),
            pltpu.VMEM((CHUNK, H_P), jnp.float32),
            pltpu.VMEM((CHUNK, H_P), jnp.float32),
            pltpu.SemaphoreType.DMA,
            pltpu.SemaphoreType.DMA,
            pltpu.SemaphoreType.DMA,
            pltpu.SemaphoreType.DMA,
            pltpu.SemaphoreType.DMA,
            pltpu.SemaphoreType.DMA,
        ],
    )


def _fnet_body(d0_ref, d1_ref, wf1, bf1, wf2, bf2, f_ref):
    filt = STEP * lax.broadcasted_iota(jnp.int32, (1, H_BP), 1).astype(
        jnp.float32)
    d_parts = []
    for b, dist_ref in ((0, d0_ref), (1, d1_ref)):
        for k in range(K_NBR):
            d_parts.append(dist_ref[:, k:k + 1])
    d_all = jnp.concatenate(d_parts, axis=0)
    g = jnp.exp(-((d_all - filt) ** 2) * INV_STEP2)
    t1 = jnp.tanh(jnp.dot(g.astype(jnp.bfloat16), wf1[...],
                          preferred_element_type=jnp.float32) + bf1[...])
    f = jnp.dot(t1.astype(jnp.bfloat16), wf2[...],
                preferred_element_type=jnp.float32) + bf2[...]
    fb = f.astype(jnp.bfloat16)
    for b in range(2):
        for k in range(K_NBR):
            j = b * K_NBR + k
            f_ref[2 * k + b] = fb[j * BN:(j + 1) * BN]


def _make_fnet(interpret=False):
    full = lambda r, c: pl.BlockSpec((r, c), lambda i: (0, 0))
    return pl.pallas_call(
        _fnet_body,
        grid=(NBN,),
        in_specs=[
            pl.BlockSpec((BN, K_NBR), lambda i: (i, 0)),
            pl.BlockSpec((BN, K_NBR), lambda i: (i + NBN, 0)),
            full(H_BP, H_A),
            full(1, H_A),
            full(H_A, H_A),
            full(1, H_A),
        ],
        out_specs=pl.BlockSpec((2 * K_NBR, BN, H_A), lambda i: (0, i, 0)),
        out_shape=jax.ShapeDtypeStruct((2 * K_NBR, N_ATOMS, H_A),
                                       jnp.bfloat16),
        interpret=interpret,
    )


def _combine_body(layer0, has_next, *refs):
    if has_next:
        (nf_ref, f_ref, x_ref, wat,
         wo1, bo1, wo2, bo2, wnext, xo_ref, ho_ref) = refs
    else:
        (nf_ref, f_ref, x_ref, wat,
         wo1, bo1, wo2, bo2, xo_ref) = refs

    cf_parts = []
    for b in range(2):
        for k in range(K_NBR):
            nf_p = nf_ref[2 * k + b, :, H_A * b:H_A * (b + 1)]
            f_p = f_ref[2 * k + b].astype(jnp.float32)
            cf_parts.append(nf_p * f_p)
    cf = jnp.concatenate(cf_parts, axis=0)
    s = jnp.sum(cf * wat[...], axis=1, keepdims=True)

    aggs = []
    for b in range(2):
        sb = jnp.concatenate(
            [s[(b * K_NBR + k) * BN:(b * K_NBR + k + 1) * BN]
             for k in range(K_NBR)], axis=1)
        mx = jnp.max(sb, axis=1, keepdims=True)
        e = jnp.exp(sb - mx)
        attn = e / jnp.sum(e, axis=1, keepdims=True)
        agg = jnp.zeros((BN, H_A), dtype=jnp.float32)
        for k in range(K_NBR):
            j = b * K_NBR + k
            agg += attn[:, k:k + 1] * cf[j * BN:(j + 1) * BN]
        aggs.append(agg)
    agg2 = jnp.concatenate(aggs, axis=0)

    t2 = jnp.tanh(jnp.dot(agg2, wo1[...],
                          preferred_element_type=jnp.float32) + bo1[...])
    out = jnp.dot(t2, wo2[...],
                  preferred_element_type=jnp.float32) + bo2[...]
    if layer0:
        xv = x_ref[...]
        x2 = jnp.concatenate([xv, xv], axis=0)
    else:
        x2 = jnp.concatenate([x_ref[:, :H_A], x_ref[:, H_A:]], axis=0)
    xn = x2 + out
    xo_ref[...] = jnp.concatenate([xn[:BN], xn[BN:]], axis=1)
    if has_next:
        hn = jnp.dot(xn, wnext[...], preferred_element_type=jnp.float32)
        ho_ref[...] = jnp.concatenate([hn[:BN], hn[BN:]], axis=1)


def _make_combine(layer0, has_next, interpret=False):
    if layer0:
        x_spec = pl.BlockSpec((BN, H_A), lambda i: (i, 0))
    else:
        x_spec = pl.BlockSpec((BN, H_P), lambda i: (i, 0))
    full = lambda r, c: pl.BlockSpec((r, c), lambda i: (0, 0))
    in_specs = [
        pl.BlockSpec((2 * K_NBR, BN, H_P), lambda i: (0, i, 0)),
        pl.BlockSpec((2 * K_NBR, BN, H_A), lambda i: (0, i, 0)),
        x_spec,
        full(1, H_A),
        full(H_A, H_A),
        full(1, H_A),
        full(H_A, H_A),
        full(1, H_A),
    ]
    out_shapes = [jax.ShapeDtypeStruct((N_ATOMS, H_P), jnp.float32)]
    out_specs = [pl.BlockSpec((BN, H_P), lambda i: (i, 0))]
    if has_next:
        in_specs.append(full(H_A, H_A))
        out_shapes.append(jax.ShapeDtypeStruct((N_ATOMS, H_P), jnp.float32))
        out_specs.append(pl.BlockSpec((BN, H_P), lambda i: (i, 0)))
    return pl.pallas_call(
        functools.partial(_combine_body, layer0, has_next),
        grid=(NBN,),
        in_specs=in_specs,
        out_specs=out_specs,
        out_shape=out_shapes,
        interpret=interpret,
    )


def _head_body(x_ref, wa, ba, wc, bc, out_ref, acc_ref):
    i = pl.program_id(0)

    @pl.when(i == 0)
    def _init():
        acc_ref[...] = jnp.zeros_like(acc_ref)

    acc_ref[...] += jnp.sum(x_ref[...], axis=0, keepdims=True)

    @pl.when(i == NBN - 1)
    def _fin():
        accv = acc_ref[...] * (1.0 / N_ATOMS)
        pooled = jnp.concatenate([accv[:, :H_A], accv[:, H_A:]], axis=0)
        emb = jnp.dot(pooled, wa[...],
                      preferred_element_type=jnp.float32) + ba[...]
        logits = jnp.dot(emb, wc[...],
                         preferred_element_type=jnp.float32) + bc[...]
        mx = jnp.max(logits, axis=1, keepdims=True)
        e = jnp.exp(logits - mx)
        out_ref[...] = e / jnp.sum(e, axis=1, keepdims=True)


def _make_head(interpret=False):
    full = lambda r, c: pl.BlockSpec((r, c), lambda i: (0, 0))
    return pl.pallas_call(
        _head_body,
        grid=(NBN,),
        in_specs=[
            pl.BlockSpec((BN, H_P), lambda i: (i, 0)),
            full(H_A, 32),
            full(1, 32),
            full(32, N_CLASSES),
            full(1, N_CLASSES),
        ],
        out_specs=full(BATCH, N_CLASSES),
        out_shape=jax.ShapeDtypeStruct((BATCH, N_CLASSES), jnp.float32),
        scratch_shapes=[pltpu.VMEM((1, H_P), jnp.float32)],
        interpret=interpret,
    )


def kernel(data, atom_embeddings, conv_params, W_amino, b_amino, W_cls, b_cls):
    dist = data[:, :, :K_NBR].reshape(ROWS, K_NBR)
    idx = data[:, :, K_NBR:].astype(jnp.int32).transpose(2, 0, 1).reshape(
        EDGES)

    h = _make_h0()(atom_embeddings, conv_params[0]['W_init'])
    sc_gather = _make_sc_gather()

    fnet = _make_fnet()
    x = atom_embeddings
    for l in range(N_LAYERS):
        p = conv_params[l]
        wf1p = jnp.concatenate(
            [p['W_f1'], jnp.zeros((H_BP - H_B, H_A), jnp.float32)],
            axis=0).astype(jnp.bfloat16)
        f = fnet(dist, dist, wf1p, p['b_f1'].reshape(1, H_A),
                 p['W_f2'].astype(jnp.bfloat16), p['b_f2'].reshape(1, H_A))
        nf = sc_gather(h, idx).reshape(2 * K_NBR, N_ATOMS, H_P)
        args = [nf, f, x, p['w_attn'].reshape(1, H_A),
                p['W_o1'], p['b_o1'].reshape(1, H_A),
                p['W_o2'], p['b_o2'].reshape(1, H_A)]
        has_next = l < N_LAYERS - 1
        if has_next:
            args.append(conv_params[l + 1]['W_init'])
            x, h = _make_combine(l == 0, True)(*args)
        else:
            (x,) = _make_combine(l == 0, False)(*args)

    return _make_head()(x, W_amino, b_amino.reshape(1, 32),
                        W_cls, b_cls.reshape(1, N_CLASSES))
